# baseline (speedup 1.0000x reference)
import functools

import jax
import jax.numpy as jnp
from jax import lax
from jax.experimental import pallas as pl
from jax.experimental.pallas import tpu as pltpu
from jax.experimental.pallas import tpu_sc as plsc

_B = 16384
_D = 64
_NEG = 5
_CB = 1024


def _sc_gather(idx, table):
    info = plsc.get_sparse_core_info()
    nw = info.num_cores * info.num_subcores
    b_per_w = _B // nw
    mesh = plsc.VectorSubcoreMesh(core_axis_name="c", subcore_axis_name="s")

    @functools.partial(
        pl.kernel,
        mesh=mesh,
        out_type=jax.ShapeDtypeStruct((_B, _D), jnp.float32),
        scratch_types=[
            pltpu.VMEM((b_per_w,), jnp.int32),
            pltpu.VMEM((b_per_w, _D), jnp.float32),
            pltpu.SemaphoreType.DMA,
            pltpu.SemaphoreType.DMA,
        ],
        compiler_params=pltpu.CompilerParams(use_tc_tiling_on_sc=True),
    )
    def k(idx_hbm, table_hbm, out_hbm, idx_v, rows_v, isem, sem):
        wid = lax.axis_index("s") * info.num_cores + lax.axis_index("c")
        base = wid * b_per_w
        pltpu.async_copy(idx_hbm.at[pl.ds(base, b_per_w)], idx_v, isem).wait()

        def fire(j, _):
            vec = idx_v[pl.ds(j * 16, 16)]
            for t in range(16):
                row = vec[t]
                pltpu.async_copy(
                    table_hbm.at[pl.ds(row, 1)],
                    rows_v.at[pl.ds(j * 16 + t, 1)],
                    sem,
                )
            return 0

        lax.fori_loop(0, b_per_w // 16, fire, 0)
        pltpu.make_async_copy(
            table_hbm.at[pl.ds(0, b_per_w)], rows_v, sem
        ).wait()
        pltpu.async_copy(rows_v, out_hbm.at[pl.ds(base, b_per_w)], isem).wait()

    return k(idx, table)


_V = 1000000
_CW = 256
_NCHUNK = 3906
_VMAIN = _NCHUNK * _CW


def _sc_transpose(table_t, tail_rm):
    info = plsc.get_sparse_core_info()
    nw = info.num_cores * info.num_subcores
    mesh = plsc.VectorSubcoreMesh(core_axis_name="c", subcore_axis_name="s")

    @functools.partial(
        pl.kernel,
        mesh=mesh,
        out_type=jax.ShapeDtypeStruct((_V, _D), jnp.float32),
        scratch_types=[
            pltpu.VMEM((_D, _CW), jnp.float32),
            pltpu.VMEM((_D, _CW), jnp.float32),
            pltpu.VMEM((_CW, _D), jnp.float32),
            pltpu.VMEM((_CW, _D), jnp.float32),
            pltpu.SemaphoreType.DMA,
            pltpu.SemaphoreType.DMA,
            pltpu.SemaphoreType.DMA,
            pltpu.SemaphoreType.DMA,
        ],
        compiler_params=pltpu.CompilerParams(
            use_tc_tiling_on_sc=True, needs_layout_passes=False
        ),
    )
    def k(tt_hbm, tail_hbm, out_hbm, buf0, buf1, tb0, tb1,
          si0, si1, so0, so1):
        wid = lax.axis_index("s") * info.num_cores + lax.axis_index("c")
        bufs = (buf0, buf1)
        tbs = (tb0, tb1)
        sis = (si0, si1)
        sos = (so0, so1)
        n_my = (_NCHUNK - wid + nw - 1) // nw
        rows_dg = [
            dg * 16 + lax.iota(jnp.int32, 16) for dg in range(4)
        ]

        def fire_in(g, b):
            c = wid + g * nw
            pltpu.async_copy(
                tt_hbm.at[:, pl.ds(c * _CW, _CW)], bufs[b], sis[b]
            )

        fire_in(0, 0)

        @pl.when(n_my > 1)
        def _():
            fire_in(1, 1)

        def do_chunk(g, b):
            c = wid + g * nw
            pltpu.make_async_copy(
                tt_hbm.at[:, pl.ds(0, _CW)], bufs[b], sis[b]
            ).wait()

            @pl.when(g >= 2)
            def _():
                pltpu.make_async_copy(
                    tbs[b], out_hbm.at[pl.ds(0, _CW)], sos[b]
                ).wait()

            def per_col(cc, _):
                cols = jnp.full((16,), cc, dtype=jnp.int32)
                for dg in range(4):
                    v = plsc.load_gather(bufs[b], [rows_dg[dg], cols])
                    tbs[b][cc, pl.ds(dg * 16, 16)] = v
                return 0

            lax.fori_loop(0, _CW, per_col, 0, unroll=2)
            pltpu.async_copy(tbs[b], out_hbm.at[pl.ds(c * _CW, _CW)], sos[b])

            @pl.when(g + 2 < n_my)
            def _():
                fire_in(g + 2, b)

        def pair(g2, _):
            for bb in range(2):
                g = g2 * 2 + bb

                @pl.when(g < n_my)
                def _():
                    do_chunk(g, bb)

            return 0

        lax.fori_loop(0, (n_my + 1) // 2, pair, 0)
        for bb in range(2):
            pltpu.make_async_copy(
                tbs[bb], out_hbm.at[pl.ds(0, _CW)], sos[bb]
            ).wait()

        @pl.when(wid == 0)
        def _():
            tvb = tb0.at[pl.ds(0, _D)]
            pltpu.async_copy(tail_hbm.at[:, :], tvb, si0).wait()
            pltpu.async_copy(tvb, out_hbm.at[pl.ds(_VMAIN, _D)], si0).wait()

    return k(table_t, tail_rm)


def _tc_loss_body(emb_ref, pos_ref, neg_ref, out_ref):
    i = pl.program_id(0)
    et = jnp.transpose(emb_ref[...])
    s = jnp.sum(pos_ref[...] * et, axis=0, keepdims=True)
    s = jnp.clip(s, -10.0, 10.0)
    acc = jnp.log1p(jnp.exp(-s))
    for kk in range(_NEG):
        ns = jnp.sum(neg_ref[kk] * et, axis=0, keepdims=True)
        ns = jnp.clip(ns, -10.0, 10.0)
        acc += jnp.log1p(jnp.exp(ns))
    part = jnp.sum(acc) * (1.0 / _B)

    @pl.when(i == 0)
    def _():
        out_ref[0, 0] = 0.0

    out_ref[0, 0] += part


def _tc_loss(emb, pos_t, neg_t):
    grid = _B // _CB
    return pl.pallas_call(
        _tc_loss_body,
        grid=(grid,),
        in_specs=[
            pl.BlockSpec((_CB, _D), lambda i: (i, 0)),
            pl.BlockSpec((_D, _CB), lambda i: (0, i)),
            pl.BlockSpec((_NEG, _D, _CB), lambda i: (0, 0, i)),
        ],
        out_specs=pl.BlockSpec(memory_space=pltpu.SMEM),
        out_shape=jax.ShapeDtypeStruct((1, 1), jnp.float32),
    )(emb, pos_t, neg_t)


def kernel(words, pos_features, neg_features, u_embeddings):
    pos_t = pos_features.T
    neg_t = jnp.transpose(neg_features, (1, 2, 0))
    tail_rm = u_embeddings[_VMAIN:]
    table_rm = _sc_transpose(u_embeddings.T, tail_rm)
    emb = _sc_gather(words, table_rm)
    loss = _tc_loss(emb, pos_t, neg_t)
    return jnp.reshape(loss, ())


# speedup vs baseline: 7.1795x; 7.1795x over previous
import functools

import jax
import jax.numpy as jnp
from jax import lax
from jax.experimental import pallas as pl
from jax.experimental.pallas import tpu as pltpu
from jax.experimental.pallas import tpu_sc as plsc

_B = 16384
_D = 64
_NEG = 5
_CB = 1024

_V = 1000000
_CW = 256
_NCHUNK = 3906
_VMAIN = _NCHUNK * _CW
_CAP = 4096
_WAVE = 96


def _sc_scan_gather(words, table_t, tail_rm):
    info = plsc.get_sparse_core_info()
    nw = info.num_cores * info.num_subcores
    mesh = plsc.VectorSubcoreMesh(core_axis_name="c", subcore_axis_name="s")

    @functools.partial(
        pl.kernel,
        mesh=mesh,
        out_type=jax.ShapeDtypeStruct((_B, _D), jnp.float32),
        scratch_types=[
            pltpu.VMEM((2048,), jnp.int32),
            pltpu.VMEM((_CAP,), jnp.int32),
            pltpu.VMEM((_CAP,), jnp.int32),
            pltpu.VMEM((_CAP,), jnp.int32),
            pltpu.VMEM((_D, _CW), jnp.float32),
            pltpu.VMEM((_D, _CW), jnp.float32),
            pltpu.VMEM((2 * _WAVE, _D), jnp.float32),
            pltpu.SMEM((2,), jnp.int32),
            pltpu.SemaphoreType.DMA,
            pltpu.SemaphoreType.DMA,
            pltpu.SemaphoreType.DMA,
            pltpu.SemaphoreType.DMA,
            pltpu.SemaphoreType.DMA,
        ],
        compiler_params=pltpu.CompilerParams(
            use_tc_tiling_on_sc=True, needs_layout_passes=False
        ),
    )
    def k(words_hbm, tt_hbm, tail_hbm, out_hbm, wbuf, widx, wb, cpack,
          buf0, buf1, stage, cnts, sw, si0, si1, so0, so1):
        wid = lax.axis_index("s") * info.num_cores + lax.axis_index("c")
        bufs = (buf0, buf1)
        sis = (si0, si1)
        sos = (so0, so1)
        iota16 = lax.iota(jnp.int32, 16)
        n_my = (_NCHUNK - wid + nw - 1) // nw

        def fire_in(g, b):
            c = wid + g * nw
            pltpu.async_copy(
                tt_hbm.at[:, pl.ds(c * _CW, _CW)], bufs[b], sis[b]
            )

        fire_in(0, 0)
        fire_in(1, 1)

        def sub(sb, cnt):
            pltpu.async_copy(
                words_hbm.at[pl.ds(sb * 2048, 2048)], wbuf, sw
            ).wait()

            def vec(i, cnt):
                w16 = wbuf[pl.ds(i * 16, 16)]
                cid = w16 >> 8
                m = (cid & (nw - 1)) == wid
                off = jnp.minimum(cnt, _CAP - 16)
                plsc.store_compressed(widx.at[pl.ds(off, 16)], w16, mask=m)
                plsc.store_compressed(
                    wb.at[pl.ds(off, 16)], sb * 2048 + i * 16 + iota16,
                    mask=m,
                )
                return cnt + plsc.all_reduce_population_count(m)[0]

            return lax.fori_loop(0, 128, vec, cnt)

        n_local = lax.fori_loop(0, 8, sub, 0)
        n_local = jnp.minimum(n_local, _CAP)
        nv = (n_local + 15) // 16
        cnts[0] = 0
        cnts[1] = 0

        def do_chunk(g, b):
            c = wid + g * nw
            pltpu.make_async_copy(
                tt_hbm.at[:, pl.ds(0, _CW)], bufs[b], sis[b]
            ).wait()

            def dw(i, _):
                pltpu.make_async_copy(
                    stage.at[pl.ds(0, 1)], out_hbm.at[pl.ds(0, 1)], sos[b]
                ).wait()
                return 0

            lax.fori_loop(0, cnts[b], dw, 0)

            def lf(v, cnt2):
                base = v * 16
                wv = widx[pl.ds(base, 16)]
                bv = wb[pl.ds(base, 16)]
                inb = (base + iota16) < n_local
                m2 = ((wv >> 8) == c) & inb
                pk = ((wv & (_CW - 1)) << 14) | bv
                off = jnp.minimum(cnt2, _CAP - 16)
                plsc.store_compressed(cpack.at[pl.ds(off, 16)], pk, mask=m2)
                return cnt2 + plsc.all_reduce_population_count(m2)[0]

            cnt2 = lax.fori_loop(0, nv, lf, 0)
            nwaves = (cnt2 + _WAVE - 1) // _WAVE

            def wave(w_i, _):
                start = w_i * _WAVE

                @pl.when(w_i >= 1)
                def _():
                    nprev = jnp.minimum(cnt2 - (w_i - 1) * _WAVE, _WAVE)

                    def dw2(i, _):
                        pltpu.make_async_copy(
                            stage.at[pl.ds(0, 1)],
                            out_hbm.at[pl.ds(0, 1)],
                            sos[b],
                        ).wait()
                        return 0

                    lax.fori_loop(0, nprev, dw2, 0)

                nthis = jnp.minimum(cnt2 - start, _WAVE)
                ngrp = (nthis + 15) // 16

                def grp(q, _):
                    goff = start + q * 16
                    pk16 = cpack[pl.ds(jnp.minimum(goff, _CAP - 16), 16)]
                    cl = (pk16 >> 14) & (_CW - 1)
                    bl = pk16 & 16383
                    rows = b * _WAVE + q * 16 + iota16
                    for d in range(_D):
                        dv = jnp.full((16,), d, dtype=jnp.int32)
                        v = plsc.load_gather(bufs[b], [dv, cl])
                        plsc.store_scatter(stage, [rows, dv], v)
                    rem = cnt2 - goff
                    for t in range(16):
                        @pl.when(t < rem)
                        def _():
                            pltpu.async_copy(
                                stage.at[pl.ds(b * _WAVE + q * 16 + t, 1)],
                                out_hbm.at[pl.ds(bl[t], 1)],
                                sos[b],
                            )
                    return 0

                lax.fori_loop(0, ngrp, grp, 0)
                return 0

            lax.fori_loop(0, nwaves, wave, 0)
            last = jnp.where(cnt2 > 0, cnt2 - (nwaves - 1) * _WAVE, 0)
            cnts[b] = last

            @pl.when(g + 2 < n_my)
            def _():
                fire_in(g + 2, b)

        def pair(g2, _):
            for bb in range(2):
                g = g2 * 2 + bb

                @pl.when(g < n_my)
                def _():
                    do_chunk(g, bb)

            return 0

        lax.fori_loop(0, (n_my + 1) // 2, pair, 0)

        for bb in range(2):
            def dwf(i, _):
                pltpu.make_async_copy(
                    stage.at[pl.ds(0, 1)], out_hbm.at[pl.ds(0, 1)], sos[bb]
                ).wait()
                return 0

            lax.fori_loop(0, cnts[bb], dwf, 0)

        def tl(v, _):
            base = v * 16
            wv = widx[pl.ds(base, 16)]
            bv = wb[pl.ds(base, 16)]
            inb = (base + iota16) < n_local
            mt = (wv >= _VMAIN) & inb

            @pl.when(plsc.all_reduce_population_count(mt)[0] > 0)
            def _():
                for t in range(16):
                    hit = (wv[t] >= _VMAIN) & ((base + t) < n_local)

                    @pl.when(hit)
                    def _():
                        pltpu.async_copy(
                            tail_hbm.at[pl.ds(wv[t] - _VMAIN, 1)],
                            out_hbm.at[pl.ds(bv[t], 1)],
                            so0,
                        ).wait()

            return 0

        lax.fori_loop(0, nv, tl, 0)

    return k(words, table_t, tail_rm)


def _tc_loss_body(emb_ref, pos_ref, neg_ref, out_ref):
    i = pl.program_id(0)
    et = jnp.transpose(emb_ref[...])
    s = jnp.sum(pos_ref[...] * et, axis=0, keepdims=True)
    s = jnp.clip(s, -10.0, 10.0)
    acc = jnp.log1p(jnp.exp(-s))
    for kk in range(_NEG):
        ns = jnp.sum(neg_ref[kk] * et, axis=0, keepdims=True)
        ns = jnp.clip(ns, -10.0, 10.0)
        acc += jnp.log1p(jnp.exp(ns))
    part = jnp.sum(acc) * (1.0 / _B)

    @pl.when(i == 0)
    def _():
        out_ref[0, 0] = 0.0

    out_ref[0, 0] += part


def _tc_loss(emb, pos_t, neg_t):
    grid = _B // _CB
    return pl.pallas_call(
        _tc_loss_body,
        grid=(grid,),
        in_specs=[
            pl.BlockSpec((_CB, _D), lambda i: (i, 0)),
            pl.BlockSpec((_D, _CB), lambda i: (0, i)),
            pl.BlockSpec((_NEG, _D, _CB), lambda i: (0, 0, i)),
        ],
        out_specs=pl.BlockSpec(memory_space=pltpu.SMEM),
        out_shape=jax.ShapeDtypeStruct((1, 1), jnp.float32),
    )(emb, pos_t, neg_t)


def kernel(words, pos_features, neg_features, u_embeddings):
    pos_t = pos_features.T
    neg_t = jnp.transpose(neg_features, (1, 2, 0))
    tail_rm = u_embeddings[_VMAIN:]
    emb = _sc_scan_gather(words, u_embeddings.T, tail_rm)
    loss = _tc_loss(emb, pos_t, neg_t)
    return jnp.reshape(loss, ())


# speedup vs baseline: 7.8511x; 1.0935x over previous
import functools

import jax
import jax.numpy as jnp
from jax import lax
from jax.experimental import pallas as pl
from jax.experimental.pallas import tpu as pltpu
from jax.experimental.pallas import tpu_sc as plsc

_B = 16384
_D = 64
_NEG = 5
_CB = 1024

_V = 1000000
_CW = 256
_NCHUNK = 3906
_VMAIN = _NCHUNK * _CW
_CAP = 4096
_WAVE = 96


def _sc_scan_gather(words, table_t, tail_rm):
    info = plsc.get_sparse_core_info()
    nw = info.num_cores * info.num_subcores
    mesh = plsc.VectorSubcoreMesh(core_axis_name="c", subcore_axis_name="s")

    @functools.partial(
        pl.kernel,
        mesh=mesh,
        out_type=jax.ShapeDtypeStruct((_B, _D), jnp.float32),
        scratch_types=[
            pltpu.VMEM((2048,), jnp.int32),
            pltpu.VMEM((_CAP,), jnp.int32),
            pltpu.VMEM((_CAP,), jnp.int32),
            pltpu.VMEM((_CAP,), jnp.int32),
            pltpu.VMEM((4096,), jnp.int32),
            pltpu.VMEM((_D, _CW), jnp.float32),
            pltpu.VMEM((_D, _CW), jnp.float32),
            pltpu.VMEM((2 * _WAVE, _D), jnp.float32),
            pltpu.SMEM((10,), jnp.int32),
            pltpu.SemaphoreType.DMA,
            pltpu.SemaphoreType.DMA,
            pltpu.SemaphoreType.DMA,
            pltpu.SemaphoreType.DMA,
            pltpu.SemaphoreType.DMA,
        ],
        compiler_params=pltpu.CompilerParams(
            use_tc_tiling_on_sc=True, needs_layout_passes=False
        ),
    )
    def k(words_hbm, tt_hbm, tail_hbm, out_hbm, wbuf, widx, wb, cpack,
          mbuf, buf0, buf1, stage, cnts, sw, si0, si1, so0, so1):
        wid = lax.axis_index("s") * info.num_cores + lax.axis_index("c")
        bufs = (buf0, buf1)
        sis = (si0, si1)
        sos = (so0, so1)
        iota16 = lax.iota(jnp.int32, 16)
        n_my = (_NCHUNK - wid + nw - 1) // nw

        def fire_in(g, b):
            c = wid + g * nw
            pltpu.async_copy(
                tt_hbm.at[:, pl.ds(c * _CW, _CW)], bufs[b], sis[b]
            )

        fire_in(0, 0)
        fire_in(1, 1)

        def sub(sb, cnt):
            pltpu.async_copy(
                words_hbm.at[pl.ds(sb * 2048, 2048)], wbuf, sw
            ).wait()

            def vec(i, cnt):
                w16 = wbuf[pl.ds(i * 16, 16)]
                cid = w16 >> 8
                m = (cid & (nw - 1)) == wid
                off = jnp.minimum(cnt, _CAP - 16)
                plsc.store_compressed(widx.at[pl.ds(off, 16)], w16, mask=m)
                plsc.store_compressed(
                    wb.at[pl.ds(off, 16)], sb * 2048 + i * 16 + iota16,
                    mask=m,
                )
                return cnt + plsc.all_reduce_population_count(m)[0]

            return lax.fori_loop(0, 128, vec, cnt)

        n_local = lax.fori_loop(0, 8, sub, 0)
        n_local = jnp.minimum(n_local, _CAP)
        nv = (n_local + 15) // 16
        cnts[0] = 0
        cnts[1] = 0

        def mb(v, mc):
            base = v * 16
            wv = widx[pl.ds(base, 16)]
            bv = wb[pl.ds(base, 16)]
            inb = (base + iota16) < n_local
            g_vec = ((wv >> 8) - wid) >> 5
            pk2 = (g_vec << 22) | ((wv & (_CW - 1)) << 14) | bv
            mm = g_vec >> 4
            nmc = []
            for m in range(8):
                msk = (mm == m) & inb
                off = m * 512 + jnp.minimum(mc[m], 512 - 16)
                plsc.store_compressed(mbuf.at[pl.ds(off, 16)], pk2, mask=msk)
                nmc.append(
                    mc[m] + plsc.all_reduce_population_count(msk)[0]
                )
            return tuple(nmc)

        mcounts = lax.fori_loop(0, nv, mb, (0, 0, 0, 0, 0, 0, 0, 0))
        for m in range(8):
            cnts[2 + m] = jnp.minimum(mcounts[m], 512)

        def do_chunk(g, b):
            c = wid + g * nw
            pltpu.make_async_copy(
                tt_hbm.at[:, pl.ds(0, _CW)], bufs[b], sis[b]
            ).wait()

            def dw(i, _):
                pltpu.make_async_copy(
                    stage.at[pl.ds(0, 1)], out_hbm.at[pl.ds(0, 1)], sos[b]
                ).wait()
                return 0

            lax.fori_loop(0, cnts[b], dw, 0)

            mseg = (g >> 4) * 512
            mcnt = cnts[2 + (g >> 4)]

            def lf(v, cnt2):
                base = v * 16
                pk2v = mbuf[pl.ds(mseg + base, 16)]
                inb = (base + iota16) < mcnt
                m2 = ((pk2v >> 22) == g) & inb
                off = jnp.minimum(cnt2, _CAP - 16)
                plsc.store_compressed(
                    cpack.at[pl.ds(off, 16)], pk2v & 0x3FFFFF, mask=m2
                )
                return cnt2 + plsc.all_reduce_population_count(m2)[0]

            cnt2 = lax.fori_loop(0, (mcnt + 15) // 16, lf, 0)
            nwaves = (cnt2 + _WAVE - 1) // _WAVE

            def wave(w_i, _):
                start = w_i * _WAVE

                @pl.when(w_i >= 1)
                def _():
                    nprev = jnp.minimum(cnt2 - (w_i - 1) * _WAVE, _WAVE)

                    def dw2(i, _):
                        pltpu.make_async_copy(
                            stage.at[pl.ds(0, 1)],
                            out_hbm.at[pl.ds(0, 1)],
                            sos[b],
                        ).wait()
                        return 0

                    lax.fori_loop(0, nprev, dw2, 0)

                nthis = jnp.minimum(cnt2 - start, _WAVE)
                ngrp = (nthis + 15) // 16

                def grp(q, _):
                    goff = start + q * 16
                    pk16 = cpack[pl.ds(jnp.minimum(goff, _CAP - 16), 16)]
                    cl = (pk16 >> 14) & (_CW - 1)
                    bl = pk16 & 16383
                    rows = b * _WAVE + q * 16 + iota16
                    for d in range(_D):
                        dv = jnp.full((16,), d, dtype=jnp.int32)
                        v = plsc.load_gather(bufs[b], [dv, cl])
                        plsc.store_scatter(stage, [rows, dv], v)
                    rem = cnt2 - goff
                    for t in range(16):
                        @pl.when(t < rem)
                        def _():
                            pltpu.async_copy(
                                stage.at[pl.ds(b * _WAVE + q * 16 + t, 1)],
                                out_hbm.at[pl.ds(bl[t], 1)],
                                sos[b],
                            )
                    return 0

                lax.fori_loop(0, ngrp, grp, 0)
                return 0

            lax.fori_loop(0, nwaves, wave, 0)
            last = jnp.where(cnt2 > 0, cnt2 - (nwaves - 1) * _WAVE, 0)
            cnts[b] = last

            @pl.when(g + 2 < n_my)
            def _():
                fire_in(g + 2, b)

        def pair(g2, _):
            for bb in range(2):
                g = g2 * 2 + bb

                @pl.when(g < n_my)
                def _():
                    do_chunk(g, bb)

            return 0

        lax.fori_loop(0, (n_my + 1) // 2, pair, 0)

        for bb in range(2):
            def dwf(i, _):
                pltpu.make_async_copy(
                    stage.at[pl.ds(0, 1)], out_hbm.at[pl.ds(0, 1)], sos[bb]
                ).wait()
                return 0

            lax.fori_loop(0, cnts[bb], dwf, 0)

        def tl(v, _):
            base = v * 16
            wv = widx[pl.ds(base, 16)]
            bv = wb[pl.ds(base, 16)]
            inb = (base + iota16) < n_local
            mt = (wv >= _VMAIN) & inb

            @pl.when(plsc.all_reduce_population_count(mt)[0] > 0)
            def _():
                for t in range(16):
                    hit = (wv[t] >= _VMAIN) & ((base + t) < n_local)

                    @pl.when(hit)
                    def _():
                        pltpu.async_copy(
                            tail_hbm.at[pl.ds(wv[t] - _VMAIN, 1)],
                            out_hbm.at[pl.ds(bv[t], 1)],
                            so0,
                        ).wait()

            return 0

        lax.fori_loop(0, nv, tl, 0)

    return k(words, table_t, tail_rm)


def _tc_loss_body(emb_ref, pos_ref, neg_ref, out_ref):
    i = pl.program_id(0)
    et = jnp.transpose(emb_ref[...])
    s = jnp.sum(pos_ref[...] * et, axis=0, keepdims=True)
    s = jnp.clip(s, -10.0, 10.0)
    acc = jnp.log1p(jnp.exp(-s))
    for kk in range(_NEG):
        ns = jnp.sum(neg_ref[kk] * et, axis=0, keepdims=True)
        ns = jnp.clip(ns, -10.0, 10.0)
        acc += jnp.log1p(jnp.exp(ns))
    part = jnp.sum(acc) * (1.0 / _B)

    @pl.when(i == 0)
    def _():
        out_ref[0, 0] = 0.0

    out_ref[0, 0] += part


def _tc_loss(emb, pos_t, neg_t):
    grid = _B // _CB
    return pl.pallas_call(
        _tc_loss_body,
        grid=(grid,),
        in_specs=[
            pl.BlockSpec((_CB, _D), lambda i: (i, 0)),
            pl.BlockSpec((_D, _CB), lambda i: (0, i)),
            pl.BlockSpec((_NEG, _D, _CB), lambda i: (0, 0, i)),
        ],
        out_specs=pl.BlockSpec(memory_space=pltpu.SMEM),
        out_shape=jax.ShapeDtypeStruct((1, 1), jnp.float32),
    )(emb, pos_t, neg_t)


def kernel(words, pos_features, neg_features, u_embeddings):
    pos_t = pos_features.T
    neg_t = jnp.transpose(neg_features, (1, 2, 0))
    tail_rm = u_embeddings[_VMAIN:]
    emb = _sc_scan_gather(words, u_embeddings.T, tail_rm)
    loss = _tc_loss(emb, pos_t, neg_t)
    return jnp.reshape(loss, ())


# speedup vs baseline: 9.0230x; 1.1493x over previous
import functools

import jax
import jax.numpy as jnp
from jax import lax
from jax.experimental import pallas as pl
from jax.experimental.pallas import tpu as pltpu
from jax.experimental.pallas import tpu_sc as plsc

_B = 16384
_D = 64
_NEG = 5
_CB = 1024

_V = 1000000
_CW = 512
_NCHUNK = 1953
_VMAIN = _NCHUNK * _CW
_CAP = 2048
_CCAP = 1024
_WAVE = 32


def _sc_scan_gather(words, table_t, tail_rm):
    info = plsc.get_sparse_core_info()
    nw = info.num_cores * info.num_subcores
    mesh = plsc.VectorSubcoreMesh(core_axis_name="c", subcore_axis_name="s")

    @functools.partial(
        pl.kernel,
        mesh=mesh,
        out_type=jax.ShapeDtypeStruct((_B, _D), jnp.float32),
        scratch_types=[
            pltpu.VMEM((2048,), jnp.int32),
            pltpu.VMEM((_CAP,), jnp.int32),
            pltpu.VMEM((_CAP,), jnp.int32),
            pltpu.VMEM((_CCAP,), jnp.int32),
            pltpu.VMEM((4096,), jnp.int32),
            pltpu.VMEM((_D, _CW), jnp.float32),
            pltpu.VMEM((_D, _CW), jnp.float32),
            pltpu.VMEM((2 * _WAVE, _D), jnp.float32),
            pltpu.SMEM((10,), jnp.int32),
            pltpu.SemaphoreType.DMA,
            pltpu.SemaphoreType.DMA,
            pltpu.SemaphoreType.DMA,
            pltpu.SemaphoreType.DMA,
            pltpu.SemaphoreType.DMA,
        ],
        compiler_params=pltpu.CompilerParams(
            use_tc_tiling_on_sc=True, needs_layout_passes=False
        ),
    )
    def k(words_hbm, tt_hbm, tail_hbm, out_hbm, wbuf, widx, wb, cpack,
          mbuf, buf0, buf1, stage, cnts, sw, si0, si1, so0, so1):
        wid = lax.axis_index("s") * info.num_cores + lax.axis_index("c")
        bufs = (buf0, buf1)
        sis = (si0, si1)
        sos = (so0, so1)
        iota16 = lax.iota(jnp.int32, 16)
        n_my = (_NCHUNK - wid + nw - 1) // nw

        def fire_in(g, b):
            c = wid + g * nw
            pltpu.async_copy(
                tt_hbm.at[:, pl.ds(c * _CW, _CW)], bufs[b], sis[b]
            )

        fire_in(0, 0)
        fire_in(1, 1)

        def sub(sb, cnt):
            pltpu.async_copy(
                words_hbm.at[pl.ds(sb * 2048, 2048)], wbuf, sw
            ).wait()

            def vec(i, cnt):
                w16 = wbuf[pl.ds(i * 16, 16)]
                cid = w16 >> 9
                m = (cid & (nw - 1)) == wid
                off = jnp.minimum(cnt, _CAP - 16)
                plsc.store_compressed(widx.at[pl.ds(off, 16)], w16, mask=m)
                plsc.store_compressed(
                    wb.at[pl.ds(off, 16)], sb * 2048 + i * 16 + iota16,
                    mask=m,
                )
                return cnt + plsc.all_reduce_population_count(m)[0]

            return lax.fori_loop(0, 128, vec, cnt)

        n_local = lax.fori_loop(0, 8, sub, 0)
        n_local = jnp.minimum(n_local, _CAP)
        nv = (n_local + 15) // 16
        cnts[0] = 0
        cnts[1] = 0

        def mb(v, mc):
            base = v * 16
            wv = widx[pl.ds(base, 16)]
            bv = wb[pl.ds(base, 16)]
            inb = (base + iota16) < n_local
            g_vec = ((wv >> 9) - wid) >> 5
            pk2 = (g_vec << 23) | ((wv & (_CW - 1)) << 14) | bv
            mm = g_vec >> 3
            nmc = []
            for m in range(8):
                msk = (mm == m) & inb
                off = m * 512 + jnp.minimum(mc[m], 512 - 16)
                plsc.store_compressed(mbuf.at[pl.ds(off, 16)], pk2, mask=msk)
                nmc.append(
                    mc[m] + plsc.all_reduce_population_count(msk)[0]
                )
            return tuple(nmc)

        mcounts = lax.fori_loop(0, nv, mb, (0, 0, 0, 0, 0, 0, 0, 0))
        for m in range(8):
            cnts[2 + m] = jnp.minimum(mcounts[m], 512)

        def do_chunk(g, b):
            c = wid + g * nw
            pltpu.make_async_copy(
                tt_hbm.at[:, pl.ds(0, _CW)], bufs[b], sis[b]
            ).wait()

            def dw(i, _):
                pltpu.make_async_copy(
                    stage.at[pl.ds(0, 1)], out_hbm.at[pl.ds(0, 1)], sos[b]
                ).wait()
                return 0

            lax.fori_loop(0, cnts[b], dw, 0)

            mseg = (g >> 3) * 512
            mcnt = cnts[2 + (g >> 3)]

            def lf(v, cnt2):
                base = v * 16
                pk2v = mbuf[pl.ds(mseg + base, 16)]
                inb = (base + iota16) < mcnt
                m2 = ((pk2v >> 23) == g) & inb
                off = jnp.minimum(cnt2, _CCAP - 16)
                plsc.store_compressed(
                    cpack.at[pl.ds(off, 16)], pk2v & 0x7FFFFF, mask=m2
                )
                return cnt2 + plsc.all_reduce_population_count(m2)[0]

            cnt2 = lax.fori_loop(0, (mcnt + 15) // 16, lf, 0)
            nwaves = (cnt2 + _WAVE - 1) // _WAVE

            def wave(w_i, _):
                start = w_i * _WAVE

                @pl.when(w_i >= 1)
                def _():
                    nprev = jnp.minimum(cnt2 - (w_i - 1) * _WAVE, _WAVE)

                    def dw2(i, _):
                        pltpu.make_async_copy(
                            stage.at[pl.ds(0, 1)],
                            out_hbm.at[pl.ds(0, 1)],
                            sos[b],
                        ).wait()
                        return 0

                    lax.fori_loop(0, nprev, dw2, 0)

                nthis = jnp.minimum(cnt2 - start, _WAVE)
                ngrp = (nthis + 15) // 16

                def grp(q, _):
                    goff = start + q * 16
                    pk16 = cpack[pl.ds(jnp.minimum(goff, _CCAP - 16), 16)]
                    cl = (pk16 >> 14) & (_CW - 1)
                    bl = pk16 & 16383
                    rows = b * _WAVE + q * 16 + iota16
                    for d in range(_D):
                        dv = jnp.full((16,), d, dtype=jnp.int32)
                        v = plsc.load_gather(bufs[b], [dv, cl])
                        plsc.store_scatter(stage, [rows, dv], v)
                    rem = cnt2 - goff
                    for t in range(16):
                        @pl.when(t < rem)
                        def _():
                            pltpu.async_copy(
                                stage.at[pl.ds(b * _WAVE + q * 16 + t, 1)],
                                out_hbm.at[pl.ds(bl[t], 1)],
                                sos[b],
                            )
                    return 0

                lax.fori_loop(0, ngrp, grp, 0)
                return 0

            lax.fori_loop(0, nwaves, wave, 0)
            last = jnp.where(cnt2 > 0, cnt2 - (nwaves - 1) * _WAVE, 0)
            cnts[b] = last

            @pl.when(g + 2 < n_my)
            def _():
                fire_in(g + 2, b)

        def pair(g2, _):
            for bb in range(2):
                g = g2 * 2 + bb

                @pl.when(g < n_my)
                def _():
                    do_chunk(g, bb)

            return 0

        lax.fori_loop(0, (n_my + 1) // 2, pair, 0)

        for bb in range(2):
            def dwf(i, _):
                pltpu.make_async_copy(
                    stage.at[pl.ds(0, 1)], out_hbm.at[pl.ds(0, 1)], sos[bb]
                ).wait()
                return 0

            lax.fori_loop(0, cnts[bb], dwf, 0)

        def tl(v, _):
            base = v * 16
            wv = widx[pl.ds(base, 16)]
            bv = wb[pl.ds(base, 16)]
            inb = (base + iota16) < n_local
            mt = (wv >= _VMAIN) & inb

            @pl.when(plsc.all_reduce_population_count(mt)[0] > 0)
            def _():
                for t in range(16):
                    hit = (wv[t] >= _VMAIN) & ((base + t) < n_local)

                    @pl.when(hit)
                    def _():
                        pltpu.async_copy(
                            tail_hbm.at[pl.ds(wv[t] - _VMAIN, 1)],
                            out_hbm.at[pl.ds(bv[t], 1)],
                            so0,
                        ).wait()

            return 0

        lax.fori_loop(0, nv, tl, 0)

    return k(words, table_t, tail_rm)


def _tc_loss_body(emb_ref, pos_ref, neg_ref, out_ref):
    i = pl.program_id(0)
    et = jnp.transpose(emb_ref[...])
    s = jnp.sum(pos_ref[...] * et, axis=0, keepdims=True)
    s = jnp.clip(s, -10.0, 10.0)
    acc = jnp.log1p(jnp.exp(-s))
    for kk in range(_NEG):
        ns = jnp.sum(neg_ref[kk] * et, axis=0, keepdims=True)
        ns = jnp.clip(ns, -10.0, 10.0)
        acc += jnp.log1p(jnp.exp(ns))
    part = jnp.sum(acc) * (1.0 / _B)

    @pl.when(i == 0)
    def _():
        out_ref[0, 0] = 0.0

    out_ref[0, 0] += part


def _tc_loss(emb, pos_t, neg_t):
    grid = _B // _CB
    return pl.pallas_call(
        _tc_loss_body,
        grid=(grid,),
        in_specs=[
            pl.BlockSpec((_CB, _D), lambda i: (i, 0)),
            pl.BlockSpec((_D, _CB), lambda i: (0, i)),
            pl.BlockSpec((_NEG, _D, _CB), lambda i: (0, 0, i)),
        ],
        out_specs=pl.BlockSpec(memory_space=pltpu.SMEM),
        out_shape=jax.ShapeDtypeStruct((1, 1), jnp.float32),
    )(emb, pos_t, neg_t)


def kernel(words, pos_features, neg_features, u_embeddings):
    pos_t = pos_features.T
    neg_t = jnp.transpose(neg_features, (1, 2, 0))
    tail_rm = u_embeddings[_VMAIN:]
    emb = _sc_scan_gather(words, u_embeddings.T, tail_rm)
    loss = _tc_loss(emb, pos_t, neg_t)
    return jnp.reshape(loss, ())


# speedup vs baseline: 9.1983x; 1.0194x over previous
import functools

import jax
import jax.numpy as jnp
from jax import lax
from jax.experimental import pallas as pl
from jax.experimental.pallas import tpu as pltpu
from jax.experimental.pallas import tpu_sc as plsc

_B = 16384
_D = 64
_NEG = 5
_CB = 1024

_V = 1000000
_CW = 512
_NCHUNK = 1953
_VMAIN = _NCHUNK * _CW
_CAP = 2048
_CCAP = 1024
_WAVE = 32


def _sc_scan_gather(words, table_t, tail_rm):
    info = plsc.get_sparse_core_info()
    nw = info.num_cores * info.num_subcores
    mesh = plsc.VectorSubcoreMesh(core_axis_name="c", subcore_axis_name="s")

    @functools.partial(
        pl.kernel,
        mesh=mesh,
        out_type=jax.ShapeDtypeStruct((_B, _D), jnp.float32),
        scratch_types=[
            pltpu.VMEM((2048,), jnp.int32),
            pltpu.VMEM((_CAP,), jnp.int32),
            pltpu.VMEM((_CAP,), jnp.int32),
            pltpu.VMEM((_CCAP,), jnp.int32),
            pltpu.VMEM((4096,), jnp.int32),
            pltpu.VMEM((_D, _CW), jnp.float32),
            pltpu.VMEM((_D, _CW), jnp.float32),
            pltpu.VMEM((2 * _WAVE, _D), jnp.float32),
            pltpu.SMEM((10,), jnp.int32),
            pltpu.SemaphoreType.DMA,
            pltpu.SemaphoreType.DMA,
            pltpu.SemaphoreType.DMA,
            pltpu.SemaphoreType.DMA,
            pltpu.SemaphoreType.DMA,
        ],
        compiler_params=pltpu.CompilerParams(
            use_tc_tiling_on_sc=True, needs_layout_passes=False
        ),
    )
    def k(words_hbm, tt_hbm, tail_hbm, out_hbm, wbuf, widx, wb, cpack,
          mbuf, buf0, buf1, stage, cnts, sw, si0, si1, so0, so1):
        wid = lax.axis_index("s") * info.num_cores + lax.axis_index("c")
        bufs = (buf0, buf1)
        sis = (si0, si1)
        sos = (so0, so1)
        iota16 = lax.iota(jnp.int32, 16)
        n_my = (_NCHUNK - wid + nw - 1) // nw

        def fire_in(g, b):
            c = wid + g * nw
            pltpu.async_copy(
                tt_hbm.at[:, pl.ds(c * _CW, _CW)], bufs[b], sis[b]
            )

        fire_in(0, 0)
        fire_in(1, 1)

        def sub(sb, cnt):
            pltpu.async_copy(
                words_hbm.at[pl.ds(sb * 2048, 2048)], wbuf, sw
            ).wait()

            def vec(i, cnt):
                w16 = wbuf[pl.ds(i * 16, 16)]
                cid = w16 >> 9
                m = (cid & (nw - 1)) == wid
                off = jnp.minimum(cnt, _CAP - 16)
                plsc.store_compressed(widx.at[pl.ds(off, 16)], w16, mask=m)
                plsc.store_compressed(
                    wb.at[pl.ds(off, 16)], sb * 2048 + i * 16 + iota16,
                    mask=m,
                )
                return cnt + plsc.all_reduce_population_count(m)[0]

            return lax.fori_loop(0, 128, vec, cnt)

        n_local = lax.fori_loop(0, 8, sub, 0)
        n_local = jnp.minimum(n_local, _CAP)
        nv = (n_local + 15) // 16
        cnts[0] = 0
        cnts[1] = 0

        def mb(v, mc):
            base = v * 16
            wv = widx[pl.ds(base, 16)]
            bv = wb[pl.ds(base, 16)]
            inb = (base + iota16) < n_local
            g_vec = ((wv >> 9) - wid) >> 5
            pk2 = (g_vec << 23) | ((wv & (_CW - 1)) << 14) | bv
            mm = g_vec >> 3
            nmc = []
            for m in range(8):
                msk = (mm == m) & inb
                off = m * 512 + jnp.minimum(mc[m], 512 - 16)
                plsc.store_compressed(mbuf.at[pl.ds(off, 16)], pk2, mask=msk)
                nmc.append(
                    mc[m] + plsc.all_reduce_population_count(msk)[0]
                )
            return tuple(nmc)

        mcounts = lax.fori_loop(0, nv, mb, (0, 0, 0, 0, 0, 0, 0, 0))
        for m in range(8):
            cnts[2 + m] = jnp.minimum(mcounts[m], 512)

        def do_chunk(g, b):
            c = wid + g * nw

            def dw(i, _):
                pltpu.make_async_copy(
                    stage.at[pl.ds(0, 1)], out_hbm.at[pl.ds(0, 1)], sos[b]
                ).wait()
                return 0

            lax.fori_loop(0, cnts[b], dw, 0)

            mseg = (g >> 3) * 512
            mcnt = cnts[2 + (g >> 3)]

            def lf(v, cnt2):
                base = v * 16
                pk2v = mbuf[pl.ds(mseg + base, 16)]
                inb = (base + iota16) < mcnt
                m2 = ((pk2v >> 23) == g) & inb
                off = jnp.minimum(cnt2, _CCAP - 16)
                plsc.store_compressed(
                    cpack.at[pl.ds(off, 16)], pk2v & 0x7FFFFF, mask=m2
                )
                return cnt2 + plsc.all_reduce_population_count(m2)[0]

            cnt2 = lax.fori_loop(0, (mcnt + 15) // 16, lf, 0)
            pltpu.make_async_copy(
                tt_hbm.at[:, pl.ds(0, _CW)], bufs[b], sis[b]
            ).wait()
            nwaves = (cnt2 + _WAVE - 1) // _WAVE

            def wave(w_i, _):
                start = w_i * _WAVE

                @pl.when(w_i >= 1)
                def _():
                    nprev = jnp.minimum(cnt2 - (w_i - 1) * _WAVE, _WAVE)

                    def dw2(i, _):
                        pltpu.make_async_copy(
                            stage.at[pl.ds(0, 1)],
                            out_hbm.at[pl.ds(0, 1)],
                            sos[b],
                        ).wait()
                        return 0

                    lax.fori_loop(0, nprev, dw2, 0)

                nthis = jnp.minimum(cnt2 - start, _WAVE)
                ngrp = (nthis + 15) // 16

                def grp(q, _):
                    goff = start + q * 16
                    pk16 = cpack[pl.ds(jnp.minimum(goff, _CCAP - 16), 16)]
                    cl = (pk16 >> 14) & (_CW - 1)
                    bl = pk16 & 16383
                    rows = b * _WAVE + q * 16 + iota16
                    for d in range(_D):
                        dv = jnp.full((16,), d, dtype=jnp.int32)
                        v = plsc.load_gather(bufs[b], [dv, cl])
                        plsc.store_scatter(stage, [rows, dv], v)
                    rem = cnt2 - goff
                    for t in range(16):
                        @pl.when(t < rem)
                        def _():
                            pltpu.async_copy(
                                stage.at[pl.ds(b * _WAVE + q * 16 + t, 1)],
                                out_hbm.at[pl.ds(bl[t], 1)],
                                sos[b],
                            )
                    return 0

                lax.fori_loop(0, ngrp, grp, 0)
                return 0

            lax.fori_loop(0, nwaves, wave, 0)
            last = jnp.where(cnt2 > 0, cnt2 - (nwaves - 1) * _WAVE, 0)
            cnts[b] = last

            @pl.when(g + 2 < n_my)
            def _():
                fire_in(g + 2, b)

        def pair(g2, _):
            for bb in range(2):
                g = g2 * 2 + bb

                @pl.when(g < n_my)
                def _():
                    do_chunk(g, bb)

            return 0

        lax.fori_loop(0, (n_my + 1) // 2, pair, 0)

        for bb in range(2):
            def dwf(i, _):
                pltpu.make_async_copy(
                    stage.at[pl.ds(0, 1)], out_hbm.at[pl.ds(0, 1)], sos[bb]
                ).wait()
                return 0

            lax.fori_loop(0, cnts[bb], dwf, 0)

        def tl(v, _):
            base = v * 16
            wv = widx[pl.ds(base, 16)]
            bv = wb[pl.ds(base, 16)]
            inb = (base + iota16) < n_local
            mt = (wv >= _VMAIN) & inb

            @pl.when(plsc.all_reduce_population_count(mt)[0] > 0)
            def _():
                for t in range(16):
                    hit = (wv[t] >= _VMAIN) & ((base + t) < n_local)

                    @pl.when(hit)
                    def _():
                        pltpu.async_copy(
                            tail_hbm.at[pl.ds(wv[t] - _VMAIN, 1)],
                            out_hbm.at[pl.ds(bv[t], 1)],
                            so0,
                        ).wait()

            return 0

        lax.fori_loop(0, nv, tl, 0)

    return k(words, table_t, tail_rm)


def _tc_loss_body(emb_ref, pos_ref, neg_ref, out_ref):
    i = pl.program_id(0)
    et = jnp.transpose(emb_ref[...])
    s = jnp.sum(pos_ref[...] * et, axis=0, keepdims=True)
    s = jnp.clip(s, -10.0, 10.0)
    acc = jnp.log1p(jnp.exp(-s))
    for kk in range(_NEG):
        ns = jnp.sum(neg_ref[kk] * et, axis=0, keepdims=True)
        ns = jnp.clip(ns, -10.0, 10.0)
        acc += jnp.log1p(jnp.exp(ns))
    part = jnp.sum(acc) * (1.0 / _B)

    @pl.when(i == 0)
    def _():
        out_ref[0, 0] = 0.0

    out_ref[0, 0] += part


def _tc_loss(emb, pos_t, neg_t):
    grid = _B // _CB
    return pl.pallas_call(
        _tc_loss_body,
        grid=(grid,),
        in_specs=[
            pl.BlockSpec((_CB, _D), lambda i: (i, 0)),
            pl.BlockSpec((_D, _CB), lambda i: (0, i)),
            pl.BlockSpec((_NEG, _D, _CB), lambda i: (0, 0, i)),
        ],
        out_specs=pl.BlockSpec(memory_space=pltpu.SMEM),
        out_shape=jax.ShapeDtypeStruct((1, 1), jnp.float32),
    )(emb, pos_t, neg_t)


def kernel(words, pos_features, neg_features, u_embeddings):
    pos_t = pos_features.T
    neg_t = jnp.transpose(neg_features, (1, 2, 0))
    tail_rm = u_embeddings[_VMAIN:]
    emb = _sc_scan_gather(words, u_embeddings.T, tail_rm)
    loss = _tc_loss(emb, pos_t, neg_t)
    return jnp.reshape(loss, ())
